# baseline (speedup 1.0000x reference)
import jax
import jax.numpy as jnp
from jax import lax
from jax.experimental import pallas as pl
from jax.experimental.pallas import tpu as pltpu

B, P, C, T = 32, 8732, 21, 12
L = 128
R = 69
PP = R * L
JT = 0.5
NPR = 3
V0, V1 = 0.1, 0.2


def _mbl_kernel(tgt_ref, conf_ref, loc_ref, dbox_ref, perm_ref,
                out_l_ref, out_c_ref, acc_ref, val_ref):
    b = pl.program_id(0)

    r_io = lax.broadcasted_iota(jnp.int32, (R, L), 0)
    l_io = lax.broadcasted_iota(jnp.int32, (R, L), 1)
    flat = r_io * L + l_io
    valid = flat < P

    cx = dbox_ref[0]
    cy = dbox_ref[1]
    w = dbox_ref[2]
    h = dbox_ref[3]
    px1 = cx - w * 0.5
    py1 = cy - h * 0.5
    px2 = cx + w * 0.5
    py2 = cy + h * 0.5
    parea = w * h

    bto = jnp.full((R, L), -1.0, dtype=jnp.float32)
    bti = jnp.zeros((R, L), dtype=jnp.int32)
    bpis = []
    txs = []
    for t in range(T):
        tx1 = tgt_ref[0, t, 0]
        ty1 = tgt_ref[0, t, 1]
        tx2 = tgt_ref[0, t, 2]
        ty2 = tgt_ref[0, t, 3]
        lbl = tgt_ref[0, t, 4]
        txs.append((tx1, ty1, tx2, ty2, lbl))
        iw = jnp.maximum(jnp.minimum(px2, tx2) - jnp.maximum(px1, tx1), 0.0)
        ih = jnp.maximum(jnp.minimum(py2, ty2) - jnp.maximum(py1, ty1), 0.0)
        inter = iw * ih
        union = (tx2 - tx1) * (ty2 - ty1) + parea - inter
        ov = inter / union
        upd = ov > bto
        bti = jnp.where(upd, t, bti)
        bto = jnp.where(upd, ov, bto)
        ovm = jnp.where(valid, ov, -1.0)
        m = jnp.max(ovm)
        bpi = jnp.min(jnp.where(ovm == m, flat, PP))
        bpis.append(bpi)

    for t in range(T):
        msk = flat == bpis[t]
        bto = jnp.where(msk, 2.0, bto)
        bti = jnp.where(msk, t, bti)

    mx1 = jnp.zeros((R, L), dtype=jnp.float32)
    my1 = jnp.zeros((R, L), dtype=jnp.float32)
    mx2 = jnp.zeros((R, L), dtype=jnp.float32)
    my2 = jnp.zeros((R, L), dtype=jnp.float32)
    lblf = jnp.zeros((R, L), dtype=jnp.float32)
    for t in range(T):
        sel = bti == t
        tx1, ty1, tx2, ty2, lbl = txs[t]
        mx1 = jnp.where(sel, tx1, mx1)
        my1 = jnp.where(sel, ty1, my1)
        mx2 = jnp.where(sel, tx2, mx2)
        my2 = jnp.where(sel, ty2, my2)
        lblf = jnp.where(sel, lbl, lblf)

    pos = jnp.logical_and(jnp.logical_not(bto < JT), valid)
    conf_lbl = jnp.where(pos, lblf.astype(jnp.int32) + 1, 0)

    gcx = ((mx1 + mx2) * 0.5 - cx) / (V0 * w)
    gcy = ((my1 + my2) * 0.5 - cy) / (V0 * h)
    gw = jnp.log((mx2 - mx1) / w) / V1
    gh = jnp.log((my2 - my1) / h) / V1

    ld = loc_ref[0]
    ldd = lax.dot_general(ld, perm_ref[...], (((1,), (0,)), ((), ())),
                          precision=lax.Precision.HIGHEST)
    ll = jnp.float32(0.0)
    for j, g in enumerate((gcx, gcy, gw, gh)):
        d = ldd[:, j * L:(j + 1) * L] - g
        ad = jnp.abs(d)
        sl1 = jnp.where(ad < 1.0, 0.5 * d * d, ad - 0.5)
        ll = ll + jnp.sum(jnp.where(pos, sl1, 0.0))

    x4 = conf_ref[0].reshape(R, L, C)
    xt = jnp.transpose(x4, (0, 2, 1))
    m3 = jnp.max(xt, axis=1, keepdims=True)
    e3 = jnp.exp(xt - m3)
    s3 = jnp.sum(e3, axis=1, keepdims=True)
    lse = m3 + jnp.log(s3)
    cio = lax.broadcasted_iota(jnp.int32, (R, C, L), 1)
    xl = jnp.sum(jnp.where(cio == conf_lbl[:, None, :], xt, 0.0), axis=1,
                 keepdims=True)
    ce = (lse - xl)[:, 0, :]

    pos_ce = jnp.sum(jnp.where(pos, ce, 0.0))

    val = jnp.where(pos, -1.0, jnp.where(valid, ce, 0.0))
    val_ref[pl.ds(b, 1), :, :] = val.reshape(1, R, L)

    @pl.when(b == 0)
    def _init():
        acc_ref[0] = ll
        acc_ref[1] = pos_ce

    @pl.when(b > 0)
    def _accum():
        acc_ref[0] = acc_ref[0] + ll
        acc_ref[1] = acc_ref[1] + pos_ce

    @pl.when(b == B - 1)
    def _finish():
        vals = val_ref[...]
        pos3 = vals < 0.0
        npos = jnp.sum(pos3.astype(jnp.int32), axis=(1, 2), keepdims=True)
        cer = jnp.maximum(vals, 0.0)
        bits = lax.bitcast_convert_type(cer, jnp.int32)
        k3 = jnp.minimum(npos * NPR, P)
        hi0 = jnp.max(bits, axis=(1, 2), keepdims=True)
        lo0 = jnp.zeros_like(hi0)

        def bs_body(_, carry):
            lo, hi = carry
            mid = lo + (hi - lo + 1) // 2
            cnt = jnp.sum((bits >= mid).astype(jnp.int32), axis=(1, 2),
                          keepdims=True)
            ok = cnt >= k3
            return (jnp.where(ok, mid, lo), jnp.where(ok, hi, mid - 1))

        lo, _ = lax.fori_loop(0, 31, bs_body, (lo0, hi0))
        v = lax.bitcast_convert_type(lo, jnp.float32)
        gt = cer > v
        cnt_gt = jnp.sum(gt.astype(jnp.float32), axis=(1, 2), keepdims=True)
        sum_gt = jnp.sum(jnp.where(gt, cer, 0.0), axis=(1, 2), keepdims=True)
        topk = sum_gt + (k3.astype(jnp.float32) - cnt_gt) * v
        n = jnp.sum(npos).astype(jnp.float32)
        out_l_ref[...] = jnp.full((1, 1), acc_ref[0] / n, dtype=jnp.float32)
        out_c_ref[...] = jnp.full(
            (1, 1), (acc_ref[1] + jnp.sum(topk)) / n, dtype=jnp.float32)


@jax.jit
def kernel(loc_data, conf_data, dbox_list, targets):
    confp = jnp.pad(conf_data, ((0, 0), (0, PP - P), (0, 0))
                    ).reshape(B, R, L * C)
    locp = jnp.pad(loc_data, ((0, 0), (0, PP - P), (0, 0))
                   ).reshape(B, R, 4 * L)
    q = jnp.arange(4 * L)
    perm = (q[:, None] == (4 * (q % L) + q // L)[None, :]).astype(jnp.float32)
    dbox4 = jnp.pad(jnp.transpose(dbox_list, (1, 0)),
                    ((0, 0), (0, PP - P)), constant_values=1.0
                    ).reshape(4, R, L)
    out_l, out_c = pl.pallas_call(
        _mbl_kernel,
        grid=(B,),
        in_specs=[
            pl.BlockSpec((1, T, 5), lambda b: (b, 0, 0),
                         memory_space=pltpu.SMEM),
            pl.BlockSpec((1, R, L * C), lambda b: (b, 0, 0)),
            pl.BlockSpec((1, R, 4 * L), lambda b: (b, 0, 0)),
            pl.BlockSpec((4, R, L), lambda b: (0, 0, 0)),
            pl.BlockSpec((4 * L, 4 * L), lambda b: (0, 0)),
        ],
        out_specs=[
            pl.BlockSpec((1, 1), lambda b: (0, 0)),
            pl.BlockSpec((1, 1), lambda b: (0, 0)),
        ],
        out_shape=[
            jax.ShapeDtypeStruct((1, 1), jnp.float32),
            jax.ShapeDtypeStruct((1, 1), jnp.float32),
        ],
        scratch_shapes=[pltpu.SMEM((2,), jnp.float32),
                        pltpu.VMEM((B, R, L), jnp.float32)],
    )(targets, confp, locp, dbox4, perm)
    return (out_l[0, 0], out_c[0, 0])


# speedup vs baseline: 3.1830x; 3.1830x over previous
import functools

import jax
import jax.numpy as jnp
from jax import lax
from jax.experimental import pallas as pl
from jax.experimental.pallas import tpu as pltpu

B, P, C, T = 32, 8732, 21, 12
L = 128
R = 69
PP = R * L
JT = 0.5
NPR = 3
V0, V1 = 0.1, 0.2


def _mbl_kernel(tgt_ref, conf_ref, loc_ref, dbox_ref, out_l_ref, out_c_ref,
                acc_ref, val_ref):
    b = pl.program_id(0)

    r_io = lax.broadcasted_iota(jnp.int32, (R, L), 0)
    l_io = lax.broadcasted_iota(jnp.int32, (R, L), 1)
    flat = r_io * L + l_io
    valid = flat < P

    cx = dbox_ref[0]
    cy = dbox_ref[1]
    w = dbox_ref[2]
    h = dbox_ref[3]
    px1 = cx - w * 0.5
    py1 = cy - h * 0.5
    px2 = cx + w * 0.5
    py2 = cy + h * 0.5
    parea = w * h

    bto = jnp.full((R, L), -1.0, dtype=jnp.float32)
    bti = jnp.zeros((R, L), dtype=jnp.int32)
    bpis = []
    txs = []
    for t in range(T):
        tx1 = tgt_ref[0, t, 0]
        ty1 = tgt_ref[0, t, 1]
        tx2 = tgt_ref[0, t, 2]
        ty2 = tgt_ref[0, t, 3]
        lbl = tgt_ref[0, t, 4]
        txs.append((tx1, ty1, tx2, ty2, lbl))
        iw = jnp.maximum(jnp.minimum(px2, tx2) - jnp.maximum(px1, tx1), 0.0)
        ih = jnp.maximum(jnp.minimum(py2, ty2) - jnp.maximum(py1, ty1), 0.0)
        inter = iw * ih
        union = (tx2 - tx1) * (ty2 - ty1) + parea - inter
        ov = inter / union
        upd = ov > bto
        bti = jnp.where(upd, t, bti)
        bto = jnp.where(upd, ov, bto)
        ovm = jnp.where(valid, ov, -1.0)
        m = jnp.max(ovm)
        bpi = jnp.min(jnp.where(ovm == m, flat, PP))
        bpis.append(bpi)

    for t in range(T):
        msk = flat == bpis[t]
        bto = jnp.where(msk, 2.0, bto)
        bti = jnp.where(msk, t, bti)

    mx1 = jnp.zeros((R, L), dtype=jnp.float32)
    my1 = jnp.zeros((R, L), dtype=jnp.float32)
    mx2 = jnp.zeros((R, L), dtype=jnp.float32)
    my2 = jnp.zeros((R, L), dtype=jnp.float32)
    lblf = jnp.zeros((R, L), dtype=jnp.float32)
    for t in range(T):
        sel = bti == t
        tx1, ty1, tx2, ty2, lbl = txs[t]
        mx1 = jnp.where(sel, tx1, mx1)
        my1 = jnp.where(sel, ty1, my1)
        mx2 = jnp.where(sel, tx2, mx2)
        my2 = jnp.where(sel, ty2, my2)
        lblf = jnp.where(sel, lbl, lblf)

    pos = jnp.logical_and(jnp.logical_not(bto < JT), valid)
    conf_lbl = jnp.where(pos, lblf.astype(jnp.int32) + 1, 0)

    gcx = ((mx1 + mx2) * 0.5 - cx) / (V0 * w)
    gcy = ((my1 + my2) * 0.5 - cy) / (V0 * h)
    gw = jnp.log((mx2 - mx1) / w) / V1
    gh = jnp.log((my2 - my1) / h) / V1
    ll = jnp.float32(0.0)
    for j, g in enumerate((gcx, gcy, gw, gh)):
        d = loc_ref[0, j] - g
        ad = jnp.abs(d)
        sl1 = jnp.where(ad < 1.0, 0.5 * d * d, ad - 0.5)
        ll = ll + jnp.sum(jnp.where(pos, sl1, 0.0))

    x = conf_ref[0]
    m = jnp.max(x, axis=0)
    s = jnp.sum(jnp.exp(x - m[None]), axis=0)
    lse = m + jnp.log(s)
    cio = lax.broadcasted_iota(jnp.int32, (C, R, L), 0)
    xl = jnp.sum(jnp.where(cio == conf_lbl[None], x, 0.0), axis=0)
    ce = lse - xl

    pos_ce = jnp.sum(jnp.where(pos, ce, 0.0))

    val = jnp.where(pos, -1.0, jnp.where(valid, ce, 0.0))
    val_ref[pl.ds(b, 1), :, :] = val.reshape(1, R, L)

    @pl.when(b == 0)
    def _init():
        acc_ref[0] = ll
        acc_ref[1] = pos_ce

    @pl.when(b > 0)
    def _accum():
        acc_ref[0] = acc_ref[0] + ll
        acc_ref[1] = acc_ref[1] + pos_ce

    @pl.when(b == B - 1)
    def _finish():
        vals = val_ref[...]
        pos3 = vals < 0.0
        npos = jnp.sum(pos3.astype(jnp.int32), axis=(1, 2), keepdims=True)
        cer = jnp.maximum(vals, 0.0)
        bits = lax.bitcast_convert_type(cer, jnp.int32)
        k3 = jnp.minimum(npos * NPR, P)
        hi0 = jnp.max(bits, axis=(1, 2), keepdims=True)
        lo0 = jnp.zeros_like(hi0)

        def bs_body(_, carry):
            lo, hi = carry
            mid = lo + (hi - lo + 1) // 2
            cnt = jnp.sum((bits >= mid).astype(jnp.int32), axis=(1, 2),
                          keepdims=True)
            ok = cnt >= k3
            return (jnp.where(ok, mid, lo), jnp.where(ok, hi, mid - 1))

        lo, _ = lax.fori_loop(0, 31, bs_body, (lo0, hi0))
        v = lax.bitcast_convert_type(lo, jnp.float32)
        gt = cer > v
        cnt_gt = jnp.sum(gt.astype(jnp.float32), axis=(1, 2), keepdims=True)
        sum_gt = jnp.sum(jnp.where(gt, cer, 0.0), axis=(1, 2), keepdims=True)
        topk = sum_gt + (k3.astype(jnp.float32) - cnt_gt) * v
        n = jnp.sum(npos).astype(jnp.float32)
        out_l_ref[...] = jnp.full((1, 1), acc_ref[0] / n, dtype=jnp.float32)
        out_c_ref[...] = jnp.full(
            (1, 1), (acc_ref[1] + jnp.sum(topk)) / n, dtype=jnp.float32)


@jax.jit
def kernel(loc_data, conf_data, dbox_list, targets):
    conf4 = jnp.pad(jnp.transpose(conf_data, (0, 2, 1)),
                    ((0, 0), (0, 0), (0, PP - P))).reshape(B, C, R, L)
    loc4 = jnp.pad(jnp.transpose(loc_data, (0, 2, 1)),
                   ((0, 0), (0, 0), (0, PP - P))).reshape(B, 4, R, L)
    dbox4 = jnp.pad(jnp.transpose(dbox_list, (1, 0)),
                    ((0, 0), (0, PP - P)), constant_values=1.0
                    ).reshape(4, R, L)

    out_l, out_c = pl.pallas_call(
        _mbl_kernel,
        grid=(B,),
        in_specs=[
            pl.BlockSpec((1, T, 5), lambda b: (b, 0, 0),
                         memory_space=pltpu.SMEM),
            pl.BlockSpec((1, C, R, L), lambda b: (b, 0, 0, 0)),
            pl.BlockSpec((1, 4, R, L), lambda b: (b, 0, 0, 0)),
            pl.BlockSpec((4, R, L), lambda b: (0, 0, 0)),
        ],
        out_specs=[
            pl.BlockSpec((1, 1), lambda b: (0, 0)),
            pl.BlockSpec((1, 1), lambda b: (0, 0)),
        ],
        out_shape=[
            jax.ShapeDtypeStruct((1, 1), jnp.float32),
            jax.ShapeDtypeStruct((1, 1), jnp.float32),
        ],
        scratch_shapes=[pltpu.SMEM((2,), jnp.float32),
                        pltpu.VMEM((B, R, L), jnp.float32)],
    )(targets, conf4, loc4, dbox4)
    return (out_l[0, 0], out_c[0, 0])


# speedup vs baseline: 3.3579x; 1.0550x over previous
import functools

import jax
import jax.numpy as jnp
from jax import lax
from jax.experimental import pallas as pl
from jax.experimental.pallas import tpu as pltpu

B, P, C, T = 32, 8732, 21, 12
L = 128
R = 69
PP = R * L
JT = 0.5
NPR = 3
V0, V1 = 0.1, 0.2


def _mbl_kernel(tgt_ref, conf_ref, loc_ref, perm_ref, dbox_ref, out_l_ref, out_c_ref,
                acc_ref, val_ref):
    b = pl.program_id(0)

    r_io = lax.broadcasted_iota(jnp.int32, (R, L), 0)
    l_io = lax.broadcasted_iota(jnp.int32, (R, L), 1)
    flat = r_io * L + l_io
    valid = flat < P

    cx = dbox_ref[0]
    cy = dbox_ref[1]
    w = dbox_ref[2]
    h = dbox_ref[3]
    px1 = cx - w * 0.5
    py1 = cy - h * 0.5
    px2 = cx + w * 0.5
    py2 = cy + h * 0.5
    parea = w * h

    bto = jnp.full((R, L), -1.0, dtype=jnp.float32)
    bti = jnp.zeros((R, L), dtype=jnp.int32)
    bpis = []
    txs = []
    for t in range(T):
        tx1 = tgt_ref[0, t, 0]
        ty1 = tgt_ref[0, t, 1]
        tx2 = tgt_ref[0, t, 2]
        ty2 = tgt_ref[0, t, 3]
        lbl = tgt_ref[0, t, 4]
        txs.append((tx1, ty1, tx2, ty2, lbl))
        iw = jnp.maximum(jnp.minimum(px2, tx2) - jnp.maximum(px1, tx1), 0.0)
        ih = jnp.maximum(jnp.minimum(py2, ty2) - jnp.maximum(py1, ty1), 0.0)
        inter = iw * ih
        union = (tx2 - tx1) * (ty2 - ty1) + parea - inter
        ov = inter / union
        upd = ov > bto
        bti = jnp.where(upd, t, bti)
        bto = jnp.where(upd, ov, bto)
        bpis.append(jnp.where(valid, ov, -1.0))

    ov3 = jnp.stack(bpis, axis=0)
    m12 = jnp.max(ov3, axis=(1, 2), keepdims=True)
    flat3 = flat[None]
    bpi12 = jnp.min(jnp.where(ov3 == m12, flat3, PP), axis=(1, 2),
                    keepdims=True)
    msk3 = flat3 == bpi12
    tio3 = lax.broadcasted_iota(jnp.int32, (T, R, L), 0)
    anym = jnp.max(jnp.where(msk3, 1, 0), axis=0) > 0
    tsel = jnp.max(jnp.where(msk3, tio3, -1), axis=0)
    bto = jnp.where(anym, 2.0, bto)
    bti = jnp.where(anym, tsel, bti)

    mx1 = jnp.zeros((R, L), dtype=jnp.float32)
    my1 = jnp.zeros((R, L), dtype=jnp.float32)
    mx2 = jnp.zeros((R, L), dtype=jnp.float32)
    my2 = jnp.zeros((R, L), dtype=jnp.float32)
    lblf = jnp.zeros((R, L), dtype=jnp.float32)
    for t in range(T):
        sel = bti == t
        tx1, ty1, tx2, ty2, lbl = txs[t]
        mx1 = jnp.where(sel, tx1, mx1)
        my1 = jnp.where(sel, ty1, my1)
        mx2 = jnp.where(sel, tx2, mx2)
        my2 = jnp.where(sel, ty2, my2)
        lblf = jnp.where(sel, lbl, lblf)

    pos = jnp.logical_and(jnp.logical_not(bto < JT), valid)
    conf_lbl = jnp.where(pos, lblf.astype(jnp.int32) + 1, 0)

    gcx = ((mx1 + mx2) * 0.5 - cx) / (V0 * w)
    gcy = ((my1 + my2) * 0.5 - cy) / (V0 * h)
    gw = jnp.log((mx2 - mx1) / w) / V1
    gh = jnp.log((my2 - my1) / h) / V1
    ld = loc_ref[0]
    ldd = lax.dot_general(ld, perm_ref[...], (((1,), (0,)), ((), ())),
                          precision=lax.Precision.HIGHEST)
    ll = jnp.float32(0.0)
    for j, g in enumerate((gcx, gcy, gw, gh)):
        d = ldd[:, j * L:(j + 1) * L] - g
        ad = jnp.abs(d)
        sl1 = jnp.where(ad < 1.0, 0.5 * d * d, ad - 0.5)
        ll = ll + jnp.sum(jnp.where(pos, sl1, 0.0))

    x = conf_ref[0]
    m = jnp.max(x, axis=0)
    s = jnp.sum(jnp.exp(x - m[None]), axis=0)
    lse = m + jnp.log(s)
    cio = lax.broadcasted_iota(jnp.int32, (C, R, L), 0)
    xl = jnp.sum(jnp.where(cio == conf_lbl[None], x, 0.0), axis=0)
    ce = lse - xl

    pos_ce = jnp.sum(jnp.where(pos, ce, 0.0))

    val = jnp.where(pos, -1.0, jnp.where(valid, ce, 0.0))
    val_ref[pl.ds(b, 1), :, :] = val.reshape(1, R, L)

    @pl.when(b == 0)
    def _init():
        acc_ref[0] = ll
        acc_ref[1] = pos_ce

    @pl.when(b > 0)
    def _accum():
        acc_ref[0] = acc_ref[0] + ll
        acc_ref[1] = acc_ref[1] + pos_ce

    @pl.when(b == B - 1)
    def _finish():
        vals = val_ref[...]
        pos3 = vals < 0.0
        npos = jnp.sum(pos3.astype(jnp.int32), axis=(1, 2), keepdims=True)
        cer = jnp.maximum(vals, 0.0)
        bits = lax.bitcast_convert_type(cer, jnp.int32)
        k3 = jnp.minimum(npos * NPR, P)
        hi0 = jnp.max(bits, axis=(1, 2), keepdims=True)
        lo0 = jnp.zeros_like(hi0)

        def bs_body(_, carry):
            lo, hi = carry
            mid = lo + (hi - lo + 1) // 2
            cnt = jnp.sum((bits >= mid).astype(jnp.int32), axis=(1, 2),
                          keepdims=True)
            ok = cnt >= k3
            return (jnp.where(ok, mid, lo), jnp.where(ok, hi, mid - 1))

        lo, _ = lax.fori_loop(0, 31, bs_body, (lo0, hi0))
        v = lax.bitcast_convert_type(lo, jnp.float32)
        gt = cer > v
        cnt_gt = jnp.sum(gt.astype(jnp.float32), axis=(1, 2), keepdims=True)
        sum_gt = jnp.sum(jnp.where(gt, cer, 0.0), axis=(1, 2), keepdims=True)
        topk = sum_gt + (k3.astype(jnp.float32) - cnt_gt) * v
        n = jnp.sum(npos).astype(jnp.float32)
        out_l_ref[...] = jnp.full((1, 1), acc_ref[0] / n, dtype=jnp.float32)
        out_c_ref[...] = jnp.full(
            (1, 1), (acc_ref[1] + jnp.sum(topk)) / n, dtype=jnp.float32)


@jax.jit
def kernel(loc_data, conf_data, dbox_list, targets):
    conf4 = jnp.pad(jnp.transpose(conf_data, (0, 2, 1)),
                    ((0, 0), (0, 0), (0, PP - P))).reshape(B, C, R, L)
    loc4 = jnp.pad(loc_data, ((0, 0), (0, PP - P), (0, 0))
                   ).reshape(B, R, 4 * L)
    q = jnp.arange(4 * L)
    perm = (q[:, None] == (4 * (q % L) + q // L)[None, :]).astype(jnp.float32)
    dbox4 = jnp.pad(jnp.transpose(dbox_list, (1, 0)),
                    ((0, 0), (0, PP - P)), constant_values=1.0
                    ).reshape(4, R, L)

    out_l, out_c = pl.pallas_call(
        _mbl_kernel,
        grid=(B,),
        in_specs=[
            pl.BlockSpec((1, T, 5), lambda b: (b, 0, 0),
                         memory_space=pltpu.SMEM),
            pl.BlockSpec((1, C, R, L), lambda b: (b, 0, 0, 0)),
            pl.BlockSpec((1, R, 4 * L), lambda b: (b, 0, 0)),
            pl.BlockSpec((4 * L, 4 * L), lambda b: (0, 0)),
            pl.BlockSpec((4, R, L), lambda b: (0, 0, 0)),
        ],
        out_specs=[
            pl.BlockSpec((1, 1), lambda b: (0, 0)),
            pl.BlockSpec((1, 1), lambda b: (0, 0)),
        ],
        out_shape=[
            jax.ShapeDtypeStruct((1, 1), jnp.float32),
            jax.ShapeDtypeStruct((1, 1), jnp.float32),
        ],
        scratch_shapes=[pltpu.SMEM((2,), jnp.float32),
                        pltpu.VMEM((B, R, L), jnp.float32)],
    )(targets, conf4, loc4, perm, dbox4)
    return (out_l[0, 0], out_c[0, 0])
